# baseline (speedup 1.0000x reference)
import functools

import jax
import jax.numpy as jnp
from jax import lax
from jax.experimental import pallas as pl
from jax.experimental.pallas import tpu as pltpu
from jax.experimental.pallas import tpu_sc as plsc

N_TOKENS = 4096
IN_DIM = 256
QUERY_DIM = 256
HALF_DIM = 128
CB = 1024
K = 32
VDIM = 64
BLK = 1024

NEG = float("-inf")


def _topk32_2(sa, sb, ncols):
    bsz = sa.shape[0]
    cols = lax.broadcasted_iota(jnp.int32, (bsz, ncols), 1)
    tcol = lax.broadcasted_iota(jnp.int32, (bsz, K), 1)

    def step(t, carry):
        sa, sb, va, ia, vb, ib = carry
        ma = jnp.max(sa, axis=1)
        mb = jnp.max(sb, axis=1)
        pa = jnp.min(jnp.where(sa == ma[:, None], cols, ncols), axis=1)
        pb = jnp.min(jnp.where(sb == mb[:, None], cols, ncols), axis=1)
        va = jnp.where(tcol == t, ma[:, None], va)
        vb = jnp.where(tcol == t, mb[:, None], vb)
        ia = jnp.where(tcol == t, pa[:, None], ia)
        ib = jnp.where(tcol == t, pb[:, None], ib)
        sa = jnp.where(cols == pa[:, None], NEG, sa)
        sb = jnp.where(cols == pb[:, None], NEG, sb)
        return sa, sb, va, ia, vb, ib

    def body(u, carry):
        for r in range(8):
            carry = step(u * 8 + r, carry)
        return carry

    zf = jnp.full((bsz, K), NEG, jnp.float32)
    zi = jnp.zeros((bsz, K), jnp.int32)
    _, _, va, ia, vb, ib = lax.fori_loop(
        0, K // 8, body, (sa, sb, zf, zi, zf, zi))
    return va, ia, vb, ib


def _tc_body(q_ref, wq_ref, bq_ref, ca_ref, cb_ref, scale_ref,
             idx_out, scores_out, w_out):
    f32 = jnp.float32
    q = jnp.dot(q_ref[:], wq_ref[:], preferred_element_type=f32) + bq_ref[:]
    qa = q[:, :HALF_DIM]
    qb = q[:, HALF_DIM:]
    dn = (((1,), (1,)), ((), ()))
    sa = lax.dot_general(qa, ca_ref[:], dn, preferred_element_type=f32)
    sb = lax.dot_general(qb, cb_ref[:], dn, preferred_element_type=f32)

    top_sa, idx_a, top_sb, idx_b = _topk32_2(sa, sb, CB)

    bsz = top_sa.shape[0]
    seg_w = [32, 16, 12, 8, 8, 8, 4, 4, 4, 4, 2, 2, 2, 2, 2, 2]
    segs = []
    for i, w in enumerate(seg_w):
        segs.append(top_sb[:, :w] + top_sa[:, i:i + 1])
    segs.append(top_sa[:, 16:K] + top_sb[:, 0:1])
    cand = jnp.concatenate(segs, axis=1)
    ncand = cand.shape[1]
    starts = []
    acc = 0
    for w in seg_w:
        starts.append(acc)
        acc += w
    tail_start = acc

    ccol = lax.broadcasted_iota(jnp.int32, (bsz, ncand), 1)
    tcol = lax.broadcasted_iota(jnp.int32, (bsz, K), 1)

    def pstep(t, carry):
        cand, vals, ps = carry
        m = jnp.max(cand, axis=1)
        p = jnp.min(jnp.where(cand == m[:, None], ccol, ncand), axis=1)
        vals = jnp.where(tcol == t, m[:, None], vals)
        ps = jnp.where(tcol == t, p[:, None], ps)
        cand = jnp.where(ccol == p[:, None], NEG, cand)
        return cand, vals, ps

    def pbody(u, carry):
        for r in range(8):
            carry = pstep(u * 8 + r, carry)
        return carry

    init = (cand, jnp.full((bsz, K), NEG, jnp.float32),
            jnp.zeros((bsz, K), jnp.int32))
    _, vals, ps = lax.fori_loop(0, K // 8, pbody, init)

    seg_i = jnp.zeros(ps.shape, jnp.int32)
    seg_start = jnp.zeros(ps.shape, jnp.int32)
    for k in range(1, len(starts)):
        ge = (ps >= starts[k]).astype(jnp.int32)
        seg_i = seg_i + ge
        seg_start = seg_start + ge * (starts[k] - starts[k - 1])
    in_tail = ps >= tail_start
    pa = jnp.where(in_tail, ps - (tail_start - 16), seg_i)
    pb = jnp.where(in_tail, 0, ps - seg_start)
    ia = jnp.zeros(ps.shape, jnp.int32)
    ib = jnp.zeros(ps.shape, jnp.int32)
    for i in range(K):
        ia = ia + jnp.where(pa == i, idx_a[:, i:i + 1], 0)
        ib = ib + jnp.where(pb == i, idx_b[:, i:i + 1], 0)
    idxs = ia * CB + ib

    e = jnp.exp(vals - vals[:, 0:1])
    w = e / jnp.sum(e, axis=1, keepdims=True) * scale_ref[0, 0]

    idx_out[:] = idxs
    scores_out[:] = vals
    w_out[:] = w


def _tc_stage(queries, Wq, bq2, ca, cb, scale2):
    grid = (N_TOKENS // BLK,)
    return pl.pallas_call(
        _tc_body,
        grid=grid,
        in_specs=[
            pl.BlockSpec((BLK, IN_DIM), lambda i: (i, 0)),
            pl.BlockSpec((IN_DIM, QUERY_DIM), lambda i: (0, 0)),
            pl.BlockSpec((1, QUERY_DIM), lambda i: (0, 0)),
            pl.BlockSpec((CB, HALF_DIM), lambda i: (0, 0)),
            pl.BlockSpec((CB, HALF_DIM), lambda i: (0, 0)),
            pl.BlockSpec(memory_space=pltpu.SMEM),
        ],
        out_specs=[
            pl.BlockSpec((BLK, K), lambda i: (i, 0)),
            pl.BlockSpec((BLK, K), lambda i: (i, 0)),
            pl.BlockSpec((BLK, K), lambda i: (i, 0)),
        ],
        out_shape=[
            jax.ShapeDtypeStruct((N_TOKENS, K), jnp.int32),
            jax.ShapeDtypeStruct((N_TOKENS, K), jnp.float32),
            jax.ShapeDtypeStruct((N_TOKENS, K), jnp.float32),
        ],
    )(queries, Wq, bq2, ca, cb, scale2)


NC = 2
NS = 16
NW = NC * NS
TOK_PER_W = N_TOKENS // NW
CHUNK_ROWS = 128
TOK_PER_CHUNK = CHUNK_ROWS // K
N_CHUNKS = TOK_PER_W // TOK_PER_CHUNK


def _sc_stage(values, idx_flat, wexp):
    mesh = plsc.VectorSubcoreMesh(core_axis_name="c", subcore_axis_name="s")

    @functools.partial(
        pl.kernel,
        mesh=mesh,
        out_type=jax.ShapeDtypeStruct((N_TOKENS, VDIM), jnp.float32),
        scratch_types=[
            pltpu.VMEM((CHUNK_ROWS,), jnp.int32),
            pltpu.VMEM((CHUNK_ROWS, VDIM), jnp.float32),
            pltpu.VMEM((CHUNK_ROWS, 16), jnp.float32),
            pltpu.VMEM((TOK_PER_CHUNK, VDIM), jnp.float32),
            pltpu.SemaphoreType.DMA,
        ],
        compiler_params=pltpu.CompilerParams(use_tc_tiling_on_sc=False),
    )
    def sc_k(vals_hbm, idx_hbm, w_hbm, out_hbm, idx_v, rows_v, w_v, out_v, sem):
        wid = lax.axis_index("s") * NC + lax.axis_index("c")

        def chunk_body(cix, _):
            off = wid * (TOK_PER_W * K) + cix * CHUNK_ROWS
            pltpu.sync_copy(idx_hbm.at[pl.ds(off, CHUNK_ROWS)], idx_v)
            cp = pltpu.async_copy(vals_hbm.at[idx_v], rows_v, sem)
            pltpu.sync_copy(w_hbm.at[pl.ds(off, CHUNK_ROWS)], w_v)
            cp.wait()
            for t in range(TOK_PER_CHUNK):
                def n_body(n, acc):
                    rr = t * K + n
                    wv = w_v[rr, :]
                    return (acc[0] + rows_v[rr, pl.ds(0, 16)] * wv,
                            acc[1] + rows_v[rr, pl.ds(16, 16)] * wv,
                            acc[2] + rows_v[rr, pl.ds(32, 16)] * wv,
                            acc[3] + rows_v[rr, pl.ds(48, 16)] * wv)
                z = jnp.zeros((16,), jnp.float32)
                a0, a1, a2, a3 = lax.fori_loop(0, K, n_body, (z, z, z, z))
                out_v[t, pl.ds(0, 16)] = a0
                out_v[t, pl.ds(16, 16)] = a1
                out_v[t, pl.ds(32, 16)] = a2
                out_v[t, pl.ds(48, 16)] = a3
            tok0 = wid * TOK_PER_W + cix * TOK_PER_CHUNK
            pltpu.sync_copy(out_v, out_hbm.at[pl.ds(tok0, TOK_PER_CHUNK)])
            return 0

        lax.fori_loop(0, N_CHUNKS, chunk_body, 0)

    return sc_k(values, idx_flat, wexp)


def kernel(queries, topk, codebook_a, codebook_b, values, value_scale, Wq, bq):
    del topk
    bq2 = jnp.reshape(bq, (1, QUERY_DIM))
    scale2 = jnp.reshape(jnp.asarray(value_scale, jnp.float32), (1, 1))
    idx, scores, w = _tc_stage(queries, Wq, bq2, codebook_a, codebook_b, scale2)

    wexp = jnp.broadcast_to(w[:, :, None], (N_TOKENS, K, 16)).reshape(N_TOKENS * K, 16)
    idx_flat = idx.reshape(N_TOKENS * K)
    weighted = _sc_stage(values, idx_flat, wexp)
    return weighted, idx, scores


# speedup vs baseline: 1.0348x; 1.0348x over previous
import functools

import jax
import jax.numpy as jnp
from jax import lax
from jax.experimental import pallas as pl
from jax.experimental.pallas import tpu as pltpu
from jax.experimental.pallas import tpu_sc as plsc

N_TOKENS = 4096
IN_DIM = 256
QUERY_DIM = 256
HALF_DIM = 128
CB = 1024
K = 32
VDIM = 64
BLK = 512

NEG = float("-inf")


def _topk32_2(sa, sb, ncols):
    bsz = sa.shape[0]
    cols = lax.broadcasted_iota(jnp.int32, (bsz, ncols), 1)
    tcol = lax.broadcasted_iota(jnp.int32, (bsz, K), 1)

    def step(t, carry):
        sa, sb, va, ia, vb, ib = carry
        ma = jnp.max(sa, axis=1)
        mb = jnp.max(sb, axis=1)
        pa = jnp.min(jnp.where(sa == ma[:, None], cols, ncols), axis=1)
        pb = jnp.min(jnp.where(sb == mb[:, None], cols, ncols), axis=1)
        va = jnp.where(tcol == t, ma[:, None], va)
        vb = jnp.where(tcol == t, mb[:, None], vb)
        ia = jnp.where(tcol == t, pa[:, None], ia)
        ib = jnp.where(tcol == t, pb[:, None], ib)
        sa = jnp.where(cols == pa[:, None], NEG, sa)
        sb = jnp.where(cols == pb[:, None], NEG, sb)
        return sa, sb, va, ia, vb, ib

    def body(u, carry):
        for r in range(4):
            carry = step(u * 4 + r, carry)
        return carry

    zf = jnp.full((bsz, K), NEG, jnp.float32)
    zi = jnp.zeros((bsz, K), jnp.int32)
    _, _, va, ia, vb, ib = lax.fori_loop(
        0, K // 4, body, (sa, sb, zf, zi, zf, zi))
    return va, ia, vb, ib


def _tc_body(q_ref, wq_ref, bq_ref, ca_ref, cb_ref, scale_ref,
             idx_out, scores_out, w_out):
    f32 = jnp.float32
    q = jnp.dot(q_ref[:], wq_ref[:], preferred_element_type=f32) + bq_ref[:]
    qa = q[:, :HALF_DIM]
    qb = q[:, HALF_DIM:]
    dn = (((1,), (1,)), ((), ()))
    sa = lax.dot_general(qa, ca_ref[:], dn, preferred_element_type=f32)
    sb = lax.dot_general(qb, cb_ref[:], dn, preferred_element_type=f32)

    top_sa, idx_a, top_sb, idx_b = _topk32_2(sa, sb, CB)

    bsz = top_sa.shape[0]
    seg_w = [32, 16, 12, 8, 8, 8, 4, 4, 4, 4, 2, 2, 2, 2, 2, 2]
    segs = []
    for i, w in enumerate(seg_w):
        segs.append(top_sb[:, :w] + top_sa[:, i:i + 1])
    segs.append(top_sa[:, 16:K] + top_sb[:, 0:1])
    cand = jnp.concatenate(segs, axis=1)
    ncand = cand.shape[1]
    starts = []
    acc = 0
    for w in seg_w:
        starts.append(acc)
        acc += w
    tail_start = acc

    ccol = lax.broadcasted_iota(jnp.int32, (bsz, ncand), 1)
    tcol = lax.broadcasted_iota(jnp.int32, (bsz, K), 1)

    def pstep(t, carry):
        cand, vals, ps = carry
        m = jnp.max(cand, axis=1)
        p = jnp.min(jnp.where(cand == m[:, None], ccol, ncand), axis=1)
        vals = jnp.where(tcol == t, m[:, None], vals)
        ps = jnp.where(tcol == t, p[:, None], ps)
        cand = jnp.where(ccol == p[:, None], NEG, cand)
        return cand, vals, ps

    def pbody(u, carry):
        for r in range(4):
            carry = pstep(u * 4 + r, carry)
        return carry

    init = (cand, jnp.full((bsz, K), NEG, jnp.float32),
            jnp.zeros((bsz, K), jnp.int32))
    _, vals, ps = lax.fori_loop(0, K // 4, pbody, init)

    seg_i = jnp.zeros(ps.shape, jnp.int32)
    seg_start = jnp.zeros(ps.shape, jnp.int32)
    for k in range(1, len(starts)):
        ge = (ps >= starts[k]).astype(jnp.int32)
        seg_i = seg_i + ge
        seg_start = seg_start + ge * (starts[k] - starts[k - 1])
    in_tail = ps >= tail_start
    pa = jnp.where(in_tail, ps - (tail_start - 16), seg_i)
    pb = jnp.where(in_tail, 0, ps - seg_start)
    ia = jnp.zeros(ps.shape, jnp.int32)
    ib = jnp.zeros(ps.shape, jnp.int32)
    for i in range(K):
        ia = ia + jnp.where(pa == i, idx_a[:, i:i + 1], 0)
        ib = ib + jnp.where(pb == i, idx_b[:, i:i + 1], 0)
    idxs = ia * CB + ib

    e = jnp.exp(vals - vals[:, 0:1])
    w = e / jnp.sum(e, axis=1, keepdims=True) * scale_ref[0, 0]

    idx_out[:] = idxs
    scores_out[:] = vals
    w_out[:] = w


def _tc_stage(queries, Wq, bq2, ca, cb, scale2):
    grid = (N_TOKENS // BLK,)
    return pl.pallas_call(
        _tc_body,
        grid=grid,
        in_specs=[
            pl.BlockSpec((BLK, IN_DIM), lambda i: (i, 0)),
            pl.BlockSpec((IN_DIM, QUERY_DIM), lambda i: (0, 0)),
            pl.BlockSpec((1, QUERY_DIM), lambda i: (0, 0)),
            pl.BlockSpec((CB, HALF_DIM), lambda i: (0, 0)),
            pl.BlockSpec((CB, HALF_DIM), lambda i: (0, 0)),
            pl.BlockSpec(memory_space=pltpu.SMEM),
        ],
        out_specs=[
            pl.BlockSpec((BLK, K), lambda i: (i, 0)),
            pl.BlockSpec((BLK, K), lambda i: (i, 0)),
            pl.BlockSpec((BLK, K), lambda i: (i, 0)),
        ],
        out_shape=[
            jax.ShapeDtypeStruct((N_TOKENS, K), jnp.int32),
            jax.ShapeDtypeStruct((N_TOKENS, K), jnp.float32),
            jax.ShapeDtypeStruct((N_TOKENS, K), jnp.float32),
        ],
    )(queries, Wq, bq2, ca, cb, scale2)


NC = 2
NS = 16
NW = NC * NS
TOK_PER_W = N_TOKENS // NW
CHUNK_ROWS = 128
TOK_PER_CHUNK = CHUNK_ROWS // K
N_CHUNKS = TOK_PER_W // TOK_PER_CHUNK


def _sc_stage(values, idx_flat, wexp):
    mesh = plsc.VectorSubcoreMesh(core_axis_name="c", subcore_axis_name="s")

    @functools.partial(
        pl.kernel,
        mesh=mesh,
        out_type=jax.ShapeDtypeStruct((N_TOKENS, VDIM), jnp.float32),
        scratch_types=[
            pltpu.VMEM((CHUNK_ROWS,), jnp.int32),
            pltpu.VMEM((CHUNK_ROWS, VDIM), jnp.float32),
            pltpu.VMEM((CHUNK_ROWS, 16), jnp.float32),
            pltpu.VMEM((TOK_PER_CHUNK, VDIM), jnp.float32),
            pltpu.SemaphoreType.DMA,
        ],
        compiler_params=pltpu.CompilerParams(use_tc_tiling_on_sc=False),
    )
    def sc_k(vals_hbm, idx_hbm, w_hbm, out_hbm, idx_v, rows_v, w_v, out_v, sem):
        wid = lax.axis_index("s") * NC + lax.axis_index("c")

        def chunk_body(cix, _):
            off = wid * (TOK_PER_W * K) + cix * CHUNK_ROWS
            pltpu.sync_copy(idx_hbm.at[pl.ds(off, CHUNK_ROWS)], idx_v)
            cp = pltpu.async_copy(vals_hbm.at[idx_v], rows_v, sem)
            pltpu.sync_copy(w_hbm.at[pl.ds(off, CHUNK_ROWS)], w_v)
            cp.wait()
            for t in range(TOK_PER_CHUNK):
                def n_body(n, acc):
                    rr = t * K + n
                    wv = w_v[rr, :]
                    return (acc[0] + rows_v[rr, pl.ds(0, 16)] * wv,
                            acc[1] + rows_v[rr, pl.ds(16, 16)] * wv,
                            acc[2] + rows_v[rr, pl.ds(32, 16)] * wv,
                            acc[3] + rows_v[rr, pl.ds(48, 16)] * wv)
                z = jnp.zeros((16,), jnp.float32)
                a0, a1, a2, a3 = lax.fori_loop(0, K, n_body, (z, z, z, z))
                out_v[t, pl.ds(0, 16)] = a0
                out_v[t, pl.ds(16, 16)] = a1
                out_v[t, pl.ds(32, 16)] = a2
                out_v[t, pl.ds(48, 16)] = a3
            tok0 = wid * TOK_PER_W + cix * TOK_PER_CHUNK
            pltpu.sync_copy(out_v, out_hbm.at[pl.ds(tok0, TOK_PER_CHUNK)])
            return 0

        lax.fori_loop(0, N_CHUNKS, chunk_body, 0)

    return sc_k(values, idx_flat, wexp)


def kernel(queries, topk, codebook_a, codebook_b, values, value_scale, Wq, bq):
    del topk
    bq2 = jnp.reshape(bq, (1, QUERY_DIM))
    scale2 = jnp.reshape(jnp.asarray(value_scale, jnp.float32), (1, 1))
    idx, scores, w = _tc_stage(queries, Wq, bq2, codebook_a, codebook_b, scale2)

    wexp = jnp.broadcast_to(w[:, :, None], (N_TOKENS, K, 16)).reshape(N_TOKENS * K, 16)
    idx_flat = idx.reshape(N_TOKENS * K)
    weighted = _sc_stage(values, idx_flat, wexp)
    return weighted, idx, scores


# speedup vs baseline: 1.0500x; 1.0146x over previous
import functools

import jax
import jax.numpy as jnp
from jax import lax
from jax.experimental import pallas as pl
from jax.experimental.pallas import tpu as pltpu
from jax.experimental.pallas import tpu_sc as plsc

N_TOKENS = 4096
IN_DIM = 256
QUERY_DIM = 256
HALF_DIM = 128
CB = 1024
K = 32
VDIM = 64
BLK = 1024

NEG = float("-inf")


def _topk32_2(sa, sb, ncols):
    bsz = sa.shape[0]
    cols = lax.broadcasted_iota(jnp.int32, (bsz, ncols), 1)
    tcol = lax.broadcasted_iota(jnp.int32, (bsz, K), 1)

    def step(t, carry):
        sa, sb, va, ia, vb, ib = carry
        ma = jnp.max(sa, axis=1)
        mb = jnp.max(sb, axis=1)
        pa = jnp.min(jnp.where(sa == ma[:, None], cols, ncols), axis=1)
        pb = jnp.min(jnp.where(sb == mb[:, None], cols, ncols), axis=1)
        va = jnp.where(tcol == t, ma[:, None], va)
        vb = jnp.where(tcol == t, mb[:, None], vb)
        ia = jnp.where(tcol == t, pa[:, None], ia)
        ib = jnp.where(tcol == t, pb[:, None], ib)
        sa = jnp.where(cols == pa[:, None], NEG, sa)
        sb = jnp.where(cols == pb[:, None], NEG, sb)
        return sa, sb, va, ia, vb, ib

    def body(u, carry):
        for r in range(4):
            carry = step(u * 4 + r, carry)
        return carry

    zf = jnp.full((bsz, K), NEG, jnp.float32)
    zi = jnp.zeros((bsz, K), jnp.int32)
    _, _, va, ia, vb, ib = lax.fori_loop(
        0, K // 4, body, (sa, sb, zf, zi, zf, zi))
    return va, ia, vb, ib


def _tc_body(q_ref, wq_ref, bq_ref, ca_ref, cb_ref, scale_ref,
             idx_out, scores_out, w_out):
    f32 = jnp.float32
    q = jnp.dot(q_ref[:], wq_ref[:], preferred_element_type=f32) + bq_ref[:]
    qa = q[:, :HALF_DIM]
    qb = q[:, HALF_DIM:]
    dn = (((1,), (1,)), ((), ()))
    sa = lax.dot_general(qa, ca_ref[:], dn, preferred_element_type=f32)
    sb = lax.dot_general(qb, cb_ref[:], dn, preferred_element_type=f32)

    top_sa, idx_a, top_sb, idx_b = _topk32_2(sa, sb, CB)

    bsz = top_sa.shape[0]
    seg_w = [32, 16, 12, 8, 8, 8, 4, 4, 4, 4, 2, 2, 2, 2, 2, 2]
    segs = []
    for i, w in enumerate(seg_w):
        segs.append(top_sb[:, :w] + top_sa[:, i:i + 1])
    segs.append(top_sa[:, 16:K] + top_sb[:, 0:1])
    cand = jnp.concatenate(segs, axis=1)
    ncand = cand.shape[1]
    starts = []
    acc = 0
    for w in seg_w:
        starts.append(acc)
        acc += w
    tail_start = acc

    ccol = lax.broadcasted_iota(jnp.int32, (bsz, ncand), 1)
    tcol = lax.broadcasted_iota(jnp.int32, (bsz, K), 1)

    def pstep(t, carry):
        cand, vals, ps = carry
        m = jnp.max(cand, axis=1)
        p = jnp.min(jnp.where(cand == m[:, None], ccol, ncand), axis=1)
        vals = jnp.where(tcol == t, m[:, None], vals)
        ps = jnp.where(tcol == t, p[:, None], ps)
        cand = jnp.where(ccol == p[:, None], NEG, cand)
        return cand, vals, ps

    def pbody(u, carry):
        for r in range(4):
            carry = pstep(u * 4 + r, carry)
        return carry

    init = (cand, jnp.full((bsz, K), NEG, jnp.float32),
            jnp.zeros((bsz, K), jnp.int32))
    _, vals, ps = lax.fori_loop(0, K // 4, pbody, init)

    seg_i = jnp.zeros(ps.shape, jnp.int32)
    seg_start = jnp.zeros(ps.shape, jnp.int32)
    for k in range(1, len(starts)):
        ge = (ps >= starts[k]).astype(jnp.int32)
        seg_i = seg_i + ge
        seg_start = seg_start + ge * (starts[k] - starts[k - 1])
    in_tail = ps >= tail_start
    pa = jnp.where(in_tail, ps - (tail_start - 16), seg_i)
    pb = jnp.where(in_tail, 0, ps - seg_start)
    ia = jnp.zeros(ps.shape, jnp.int32)
    ib = jnp.zeros(ps.shape, jnp.int32)
    for i in range(K):
        ia = ia + jnp.where(pa == i, idx_a[:, i:i + 1], 0)
        ib = ib + jnp.where(pb == i, idx_b[:, i:i + 1], 0)
    idxs = ia * CB + ib

    e = jnp.exp(vals - vals[:, 0:1])
    w = e / jnp.sum(e, axis=1, keepdims=True) * scale_ref[0, 0]

    idx_out[:] = idxs
    scores_out[:] = vals
    w_out[:] = w


def _tc_stage(queries, Wq, bq2, ca, cb, scale2):
    grid = (N_TOKENS // BLK,)
    return pl.pallas_call(
        _tc_body,
        grid=grid,
        in_specs=[
            pl.BlockSpec((BLK, IN_DIM), lambda i: (i, 0)),
            pl.BlockSpec((IN_DIM, QUERY_DIM), lambda i: (0, 0)),
            pl.BlockSpec((1, QUERY_DIM), lambda i: (0, 0)),
            pl.BlockSpec((CB, HALF_DIM), lambda i: (0, 0)),
            pl.BlockSpec((CB, HALF_DIM), lambda i: (0, 0)),
            pl.BlockSpec(memory_space=pltpu.SMEM),
        ],
        out_specs=[
            pl.BlockSpec((BLK, K), lambda i: (i, 0)),
            pl.BlockSpec((BLK, K), lambda i: (i, 0)),
            pl.BlockSpec((BLK, K), lambda i: (i, 0)),
        ],
        out_shape=[
            jax.ShapeDtypeStruct((N_TOKENS, K), jnp.int32),
            jax.ShapeDtypeStruct((N_TOKENS, K), jnp.float32),
            jax.ShapeDtypeStruct((N_TOKENS, K), jnp.float32),
        ],
    )(queries, Wq, bq2, ca, cb, scale2)


NC = 2
NS = 16
NW = NC * NS
TOK_PER_W = N_TOKENS // NW
CHUNK_ROWS = 128
TOK_PER_CHUNK = CHUNK_ROWS // K
N_CHUNKS = TOK_PER_W // TOK_PER_CHUNK


def _sc_stage(values, idx_flat, wexp):
    mesh = plsc.VectorSubcoreMesh(core_axis_name="c", subcore_axis_name="s")

    @functools.partial(
        pl.kernel,
        mesh=mesh,
        out_type=jax.ShapeDtypeStruct((N_TOKENS, VDIM), jnp.float32),
        scratch_types=[
            pltpu.VMEM((CHUNK_ROWS,), jnp.int32),
            pltpu.VMEM((CHUNK_ROWS, VDIM), jnp.float32),
            pltpu.VMEM((CHUNK_ROWS, 16), jnp.float32),
            pltpu.VMEM((TOK_PER_CHUNK, VDIM), jnp.float32),
            pltpu.SemaphoreType.DMA,
        ],
        compiler_params=pltpu.CompilerParams(use_tc_tiling_on_sc=False),
    )
    def sc_k(vals_hbm, idx_hbm, w_hbm, out_hbm, idx_v, rows_v, w_v, out_v, sem):
        wid = lax.axis_index("s") * NC + lax.axis_index("c")

        def chunk_body(cix, _):
            off = wid * (TOK_PER_W * K) + cix * CHUNK_ROWS
            pltpu.sync_copy(idx_hbm.at[pl.ds(off, CHUNK_ROWS)], idx_v)
            cp = pltpu.async_copy(vals_hbm.at[idx_v], rows_v, sem)
            pltpu.sync_copy(w_hbm.at[pl.ds(off, CHUNK_ROWS)], w_v)
            cp.wait()
            for t in range(TOK_PER_CHUNK):
                def n_body(n, acc):
                    rr = t * K + n
                    wv = w_v[rr, :]
                    return (acc[0] + rows_v[rr, pl.ds(0, 16)] * wv,
                            acc[1] + rows_v[rr, pl.ds(16, 16)] * wv,
                            acc[2] + rows_v[rr, pl.ds(32, 16)] * wv,
                            acc[3] + rows_v[rr, pl.ds(48, 16)] * wv)
                z = jnp.zeros((16,), jnp.float32)
                a0, a1, a2, a3 = lax.fori_loop(0, K, n_body, (z, z, z, z))
                out_v[t, pl.ds(0, 16)] = a0
                out_v[t, pl.ds(16, 16)] = a1
                out_v[t, pl.ds(32, 16)] = a2
                out_v[t, pl.ds(48, 16)] = a3
            tok0 = wid * TOK_PER_W + cix * TOK_PER_CHUNK
            pltpu.sync_copy(out_v, out_hbm.at[pl.ds(tok0, TOK_PER_CHUNK)])
            return 0

        lax.fori_loop(0, N_CHUNKS, chunk_body, 0)

    return sc_k(values, idx_flat, wexp)


def kernel(queries, topk, codebook_a, codebook_b, values, value_scale, Wq, bq):
    del topk
    bq2 = jnp.reshape(bq, (1, QUERY_DIM))
    scale2 = jnp.reshape(jnp.asarray(value_scale, jnp.float32), (1, 1))
    idx, scores, w = _tc_stage(queries, Wq, bq2, codebook_a, codebook_b, scale2)

    wexp = jnp.broadcast_to(w[:, :, None], (N_TOKENS, K, 16)).reshape(N_TOKENS * K, 16)
    idx_flat = idx.reshape(N_TOKENS * K)
    weighted = _sc_stage(values, idx_flat, wexp)
    return weighted, idx, scores


# speedup vs baseline: 1.0667x; 1.0159x over previous
import functools

import jax
import jax.numpy as jnp
from jax import lax
from jax.experimental import pallas as pl
from jax.experimental.pallas import tpu as pltpu
from jax.experimental.pallas import tpu_sc as plsc

N_TOKENS = 4096
IN_DIM = 256
QUERY_DIM = 256
HALF_DIM = 128
CB = 1024
K = 32
VDIM = 64
BLK = 1024

NEG = float("-inf")


def _topk32_2(sa, sb, ncols):
    bsz = sa.shape[0]
    cols = lax.broadcasted_iota(jnp.int32, (bsz, ncols), 1)
    tcol = lax.broadcasted_iota(jnp.int32, (bsz, K), 1)

    def step(t, carry):
        sa, sb, va, ia, vb, ib = carry
        ma = jnp.max(sa, axis=1)
        mb = jnp.max(sb, axis=1)
        pa = jnp.min(jnp.where(sa == ma[:, None], cols, ncols), axis=1)
        pb = jnp.min(jnp.where(sb == mb[:, None], cols, ncols), axis=1)
        va = jnp.where(tcol == t, ma[:, None], va)
        vb = jnp.where(tcol == t, mb[:, None], vb)
        ia = jnp.where(tcol == t, pa[:, None], ia)
        ib = jnp.where(tcol == t, pb[:, None], ib)
        sa = jnp.where(cols == pa[:, None], NEG, sa)
        sb = jnp.where(cols == pb[:, None], NEG, sb)
        return sa, sb, va, ia, vb, ib

    def body(u, carry):
        for r in range(4):
            carry = step(u * 4 + r, carry)
        return carry

    zf = jnp.full((bsz, K), NEG, jnp.float32)
    zi = jnp.zeros((bsz, K), jnp.int32)
    _, _, va, ia, vb, ib = lax.fori_loop(
        0, K // 4, body, (sa, sb, zf, zi, zf, zi))
    return va, ia, vb, ib


def _tc_body(q_ref, wq_ref, bq_ref, ca_ref, cb_ref, scale_ref,
             idx_out, scores_out, w_out):
    f32 = jnp.float32
    q = jnp.dot(q_ref[:], wq_ref[:], preferred_element_type=f32) + bq_ref[:]
    qa = q[:, :HALF_DIM]
    qb = q[:, HALF_DIM:]
    dn = (((1,), (1,)), ((), ()))
    sa = lax.dot_general(qa, ca_ref[:], dn, preferred_element_type=f32)
    sb = lax.dot_general(qb, cb_ref[:], dn, preferred_element_type=f32)

    top_sa, idx_a, top_sb, idx_b = _topk32_2(sa, sb, CB)

    bsz = top_sa.shape[0]
    seg_w = [32, 16, 12, 8, 8, 8, 4, 4, 4, 4, 2, 2, 2, 2, 2, 2]
    segs = []
    for i, w in enumerate(seg_w):
        segs.append(top_sb[:, :w] + top_sa[:, i:i + 1])
    segs.append(top_sa[:, 16:K] + top_sb[:, 0:1])
    cand = jnp.concatenate(segs, axis=1)
    ncand = cand.shape[1]
    starts = []
    acc = 0
    for w in seg_w:
        starts.append(acc)
        acc += w
    tail_start = acc

    ccol = lax.broadcasted_iota(jnp.int32, (bsz, ncand), 1)
    tcol = lax.broadcasted_iota(jnp.int32, (bsz, K), 1)

    def pstep(t, carry):
        cand, vals, ps = carry
        m = jnp.max(cand, axis=1)
        p = jnp.min(jnp.where(cand == m[:, None], ccol, ncand), axis=1)
        vals = jnp.where(tcol == t, m[:, None], vals)
        ps = jnp.where(tcol == t, p[:, None], ps)
        cand = jnp.where(ccol == p[:, None], NEG, cand)
        return cand, vals, ps

    def pbody(u, carry):
        for r in range(4):
            carry = pstep(u * 4 + r, carry)
        return carry

    init = (cand, jnp.full((bsz, K), NEG, jnp.float32),
            jnp.zeros((bsz, K), jnp.int32))
    _, vals, ps = lax.fori_loop(0, K // 4, pbody, init)

    seg_i = jnp.zeros(ps.shape, jnp.int32)
    seg_start = jnp.zeros(ps.shape, jnp.int32)
    for k in range(1, len(starts)):
        ge = (ps >= starts[k]).astype(jnp.int32)
        seg_i = seg_i + ge
        seg_start = seg_start + ge * (starts[k] - starts[k - 1])
    in_tail = ps >= tail_start
    pa = jnp.where(in_tail, ps - (tail_start - 16), seg_i)
    pb = jnp.where(in_tail, 0, ps - seg_start)
    ia = jnp.zeros(ps.shape, jnp.int32)
    ib = jnp.zeros(ps.shape, jnp.int32)
    for i in range(K):
        ia = ia + jnp.where(pa == i, idx_a[:, i:i + 1], 0)
        ib = ib + jnp.where(pb == i, idx_b[:, i:i + 1], 0)
    idxs = ia * CB + ib

    e = jnp.exp(vals - vals[:, 0:1])
    w = e / jnp.sum(e, axis=1, keepdims=True) * scale_ref[0, 0]

    idx_out[:] = idxs
    scores_out[:] = vals
    w_out[:] = w


def _tc_stage(queries, Wq, bq2, ca, cb, scale2):
    grid = (N_TOKENS // BLK,)
    return pl.pallas_call(
        _tc_body,
        grid=grid,
        in_specs=[
            pl.BlockSpec((BLK, IN_DIM), lambda i: (i, 0)),
            pl.BlockSpec((IN_DIM, QUERY_DIM), lambda i: (0, 0)),
            pl.BlockSpec((1, QUERY_DIM), lambda i: (0, 0)),
            pl.BlockSpec((CB, HALF_DIM), lambda i: (0, 0)),
            pl.BlockSpec((CB, HALF_DIM), lambda i: (0, 0)),
            pl.BlockSpec(memory_space=pltpu.SMEM),
        ],
        out_specs=[
            pl.BlockSpec((BLK, K), lambda i: (i, 0)),
            pl.BlockSpec((BLK, K), lambda i: (i, 0)),
            pl.BlockSpec((BLK, K), lambda i: (i, 0)),
        ],
        out_shape=[
            jax.ShapeDtypeStruct((N_TOKENS, K), jnp.int32),
            jax.ShapeDtypeStruct((N_TOKENS, K), jnp.float32),
            jax.ShapeDtypeStruct((N_TOKENS, K), jnp.float32),
        ],
    )(queries, Wq, bq2, ca, cb, scale2)


NC = 2
NS = 16
NW = NC * NS
TOK_PER_W = N_TOKENS // NW
CHUNK_ROWS = 128
TOK_PER_CHUNK = CHUNK_ROWS // K
N_CHUNKS = TOK_PER_W // TOK_PER_CHUNK


def _sc_stage(values, idx_flat, wexp):
    mesh = plsc.VectorSubcoreMesh(core_axis_name="c", subcore_axis_name="s")

    @functools.partial(
        pl.kernel,
        mesh=mesh,
        out_type=jax.ShapeDtypeStruct((N_TOKENS, VDIM), jnp.float32),
        scratch_types=[
            pltpu.VMEM((CHUNK_ROWS,), jnp.int32),
            pltpu.VMEM((CHUNK_ROWS,), jnp.int32),
            pltpu.VMEM((CHUNK_ROWS, VDIM), jnp.float32),
            pltpu.VMEM((CHUNK_ROWS, VDIM), jnp.float32),
            pltpu.VMEM((2 * CHUNK_ROWS, 16), jnp.float32),
            pltpu.VMEM((2 * TOK_PER_CHUNK, VDIM), jnp.float32),
            pltpu.SemaphoreType.DMA,
        ],
        compiler_params=pltpu.CompilerParams(use_tc_tiling_on_sc=False),
    )
    def sc_k(vals_hbm, idx_hbm, w_hbm, out_hbm, idx_v0, idx_v1,
             rows_v0, rows_v1, w_v, out_v, sem):
        wid = lax.axis_index("s") * NC + lax.axis_index("c")

        def chunk_body(cix, _):
            off = wid * (TOK_PER_W * K) + cix * (2 * CHUNK_ROWS)
            pltpu.sync_copy(idx_hbm.at[pl.ds(off, CHUNK_ROWS)], idx_v0)
            cp0 = pltpu.async_copy(vals_hbm.at[idx_v0], rows_v0, sem)
            pltpu.sync_copy(
                idx_hbm.at[pl.ds(off + CHUNK_ROWS, CHUNK_ROWS)], idx_v1)
            cp1 = pltpu.async_copy(vals_hbm.at[idx_v1], rows_v1, sem)
            pltpu.sync_copy(w_hbm.at[pl.ds(off, 2 * CHUNK_ROWS)], w_v)
            cp0.wait()
            cp1.wait()
            for half, rows_v in ((0, rows_v0), (1, rows_v1)):
                for t in range(TOK_PER_CHUNK):
                    def n_body(n, acc):
                        rr = t * K + n
                        wv = w_v[half * CHUNK_ROWS + rr, :]
                        return (acc[0] + rows_v[rr, pl.ds(0, 16)] * wv,
                                acc[1] + rows_v[rr, pl.ds(16, 16)] * wv,
                                acc[2] + rows_v[rr, pl.ds(32, 16)] * wv,
                                acc[3] + rows_v[rr, pl.ds(48, 16)] * wv)
                    z = jnp.zeros((16,), jnp.float32)
                    a0, a1, a2, a3 = lax.fori_loop(0, K, n_body, (z, z, z, z))
                    tt = half * TOK_PER_CHUNK + t
                    out_v[tt, pl.ds(0, 16)] = a0
                    out_v[tt, pl.ds(16, 16)] = a1
                    out_v[tt, pl.ds(32, 16)] = a2
                    out_v[tt, pl.ds(48, 16)] = a3
            tok0 = wid * TOK_PER_W + cix * (2 * TOK_PER_CHUNK)
            pltpu.sync_copy(out_v, out_hbm.at[pl.ds(tok0, 2 * TOK_PER_CHUNK)])
            return 0

        lax.fori_loop(0, N_CHUNKS // 2, chunk_body, 0)

    return sc_k(values, idx_flat, wexp)


def kernel(queries, topk, codebook_a, codebook_b, values, value_scale, Wq, bq):
    del topk
    bq2 = jnp.reshape(bq, (1, QUERY_DIM))
    scale2 = jnp.reshape(jnp.asarray(value_scale, jnp.float32), (1, 1))
    idx, scores, w = _tc_stage(queries, Wq, bq2, codebook_a, codebook_b, scale2)

    wexp = jnp.broadcast_to(w[:, :, None], (N_TOKENS, K, 16)).reshape(N_TOKENS * K, 16)
    idx_flat = idx.reshape(N_TOKENS * K)
    weighted = _sc_stage(values, idx_flat, wexp)
    return weighted, idx, scores
